# baseline (speedup 1.0000x reference)
import functools

import jax
import jax.numpy as jnp
from jax import lax
from jax.experimental import pallas as pl
from jax.experimental.pallas import tpu as pltpu
import jax.experimental.pallas.tpu_sc as plsc

_N = 10000
_E = 320000
_D = 128
_G = 128
_NLAYERS = 3

_NC = 2
_NS = 16
_CHUNK = 128
_NCHUNK = _E // 128
_NPAD = 10240
_RPT = _NPAD // _NS
_DSUB = 64
_EPAD = 327680
_CHT = _EPAD // _CHUNK
_MC = 64
_MCH = _EPAD // _MC
_CPW = _MCH // (_NC * _NS)


def _zero_vmem_2d(ref, rows, cols):
  def body(r, carry):
    for k in range(cols // 16):
      ref[r, pl.ds(k * 16, 16)] = jnp.zeros((16,), jnp.float32)
    return carry
  lax.fori_loop(0, rows, body, 0)


def _n_iters(w, nworkers):
  full = _NCHUNK // nworkers
  rem = _NCHUNK - full * nworkers
  return full + jnp.where(w < rem, 1, 0)


def _msgpass_body(x_hbm, srcr_hbm, dstr_hbm, out_hbm, *scr):
  rows = list(scr[0:4])
  didx = list(scr[4:8])
  sidx = scr[8]
  gsem = list(scr[9:13])
  dsem = list(scr[13:17])
  tsem = list(scr[17:19])
  acc = scr[19]

  c = lax.axis_index("c")
  s = lax.axis_index("s")
  w = c * _NS + s

  _zero_vmem_2d(rows[0], _MC, _D)
  for k in range(_RPT // _MC):
    pltpu.sync_copy(rows[0], acc.at[pl.ds(s * _RPT + k * _MC, _MC)])

  pltpu.sync_copy(srcr_hbm.at[pl.ds(w * _CPW, _CPW)], sidx)

  def gstart(j, q):
    pltpu.make_async_copy(x_hbm.at[sidx.at[j]], rows[q], gsem[q]).start()

  def gwait(j, q):
    pltpu.make_async_copy(x_hbm.at[sidx.at[j]], rows[q], gsem[q]).wait()

  def dstart(j, q):
    pltpu.make_async_copy(
        dstr_hbm.at[pl.ds(w * _CPW + j, 1)], didx[q], dsem[q]).start()

  def dwait(j, q):
    pltpu.make_async_copy(
        dstr_hbm.at[pl.ds(w * _CPW + j, 1)], didx[q], dsem[q]).wait()

  def sstart(q, q2):
    pltpu.async_copy(rows[q], acc.at[didx[q].at[0]], tsem[q2], add=True)

  def swait(q, q2):
    pltpu.make_async_copy(rows[q], acc.at[didx[q].at[0]], tsem[q2]).wait()

  for q in range(2):
    dstart(q, q)
    gstart(q, q)

  plsc.subcore_barrier()

  def iteration(j, q, first, last):
    gwait(j, q)
    dwait(j, q)
    if not first:
      swait((q + 2) % 4, q % 2)
    sstart(q, q % 2)
    if not last:
      gstart(j + 2, (q + 2) % 4)
      dstart(j + 2, (q + 2) % 4)

  for j in range(4):
    iteration(j, j, j < 2, False)

  def body(p, carry):
    for q in range(4):
      iteration(4 * p + q, q, False, False)
    return carry

  lax.fori_loop(1, _CPW // 4 - 1, body, 0)

  for j in range(_CPW - 4, _CPW):
    iteration(j, j % 4, False, j >= _CPW - 2)
  for j in range(_CPW - 2, _CPW):
    swait(j % 4, j % 2)

  plsc.subcore_barrier()
  for k in range(_RPT // _MC):
    pltpu.sync_copy(acc.at[pl.ds(s * _RPT + k * _MC, _MC)], rows[k % 4])
    pltpu.sync_copy(
        rows[k % 4],
        out_hbm.at[pl.ds((c * _NS + s) * _RPT + k * _MC, _MC)])


@functools.lru_cache(maxsize=None)
def _msgpass_call():
  mesh = plsc.VectorSubcoreMesh(
      core_axis_name="c", subcore_axis_name="s", num_cores=_NC)
  scratch = (
      [pltpu.VMEM((_MC, _D), jnp.float32)] * 4
      + [pltpu.VMEM((1, _MC), jnp.int32)] * 4
      + [pltpu.VMEM((_CPW, _MC), jnp.int32)]
      + [pltpu.SemaphoreType.DMA] * 10
      + [pltpu.VMEM_SHARED((_NPAD, _D), jnp.float32)]
  )
  return pl.kernel(
      _msgpass_body,
      out_type=jax.ShapeDtypeStruct((_NC * _NPAD, _D), jnp.float32),
      mesh=mesh,
      scratch_types=scratch,
      compiler_params=pltpu.CompilerParams(use_tc_tiling_on_sc=False),
  )


def _deg_body(srcd_hbm, dstr_hbm, oin_hbm, oout_hbm,
              sidx, didx, hin, hout):
  c = lax.axis_index("c")
  s = lax.axis_index("s")
  w = c * _NS + s

  def zbody(r, carry):
    hin[pl.ds(r * 16, 16)] = jnp.zeros((16,), jnp.float32)
    hout[pl.ds(r * 16, 16)] = jnp.zeros((16,), jnp.float32)
    return carry
  lax.fori_loop(0, _NPAD // 16, zbody, 0)

  cpw = _CHT // 32
  pltpu.sync_copy(srcd_hbm.at[pl.ds(w * cpw, cpw)], sidx)
  pltpu.sync_copy(dstr_hbm.at[pl.ds(w * cpw, cpw)], didx)

  ones16 = jnp.ones((16,), jnp.float32)

  def step(j, carry):
    for k in range(_CHUNK // 16):
      plsc.addupdate_scatter(hout, [sidx[j, pl.ds(k * 16, 16)]], ones16)
      plsc.addupdate_scatter(hin, [didx[j, pl.ds(k * 16, 16)]], ones16)
    return carry

  lax.fori_loop(0, cpw, step, 0)

  pltpu.sync_copy(hin, oin_hbm.at[w])
  pltpu.sync_copy(hout, oout_hbm.at[w])


@functools.lru_cache(maxsize=None)
def _deg_call():
  mesh = plsc.VectorSubcoreMesh(
      core_axis_name="c", subcore_axis_name="s", num_cores=_NC)
  return pl.kernel(
      _deg_body,
      out_type=(
          jax.ShapeDtypeStruct((2 * _NS, _NPAD), jnp.float32),
          jax.ShapeDtypeStruct((2 * _NS, _NPAD), jnp.float32),
      ),
      mesh=mesh,
      scratch_types=[
          pltpu.VMEM((_CHT // 32, _CHUNK), jnp.int32),
          pltpu.VMEM((_CHT // 32, _CHUNK), jnp.int32),
          pltpu.VMEM((_NPAD,), jnp.float32),
          pltpu.VMEM((_NPAD,), jnp.float32),
      ],
      compiler_params=pltpu.CompilerParams(needs_layout_passes=False),
  )


def _prep_body(nodes, wemb, bemb, din_p, dout_p,
               h0_ref, xcat_ref, inrs_ref, outrs_ref):
  h0 = jnp.dot(nodes[...], wemb[...],
               preferred_element_type=jnp.float32) + bemb[...]
  din = jnp.sum(din_p[...], axis=0)[0:_N, None]
  dout = jnp.sum(dout_p[...], axis=0)[0:_N, None]
  inrs = lax.rsqrt(jnp.maximum(din, 1.0))
  outrs = lax.rsqrt(jnp.maximum(dout, 1.0))
  h0_ref[...] = h0
  xcat_ref[...] = h0 * outrs
  inrs_ref[...] = inrs
  outrs_ref[...] = outrs


@functools.lru_cache(maxsize=None)
def _prep_call():
  return pl.pallas_call(
      _prep_body,
      out_shape=(
          jax.ShapeDtypeStruct((_N, _D), jnp.float32),
          jax.ShapeDtypeStruct((_N, _D), jnp.float32),
          jax.ShapeDtypeStruct((_N, 1), jnp.float32),
          jax.ShapeDtypeStruct((_N, 1), jnp.float32),
      ),
      compiler_params=pltpu.CompilerParams(
          vmem_limit_bytes=100 * 1024 * 1024),
  )


def _layer_body(p, inrs, hin, w, b, snorm, gamma, beta, outrs,
                hout_ref, xcat_ref):
  agg = (p[0:_N] + p[_NPAD:_NPAD + _N]) * inrs[...]
  x = jnp.dot(agg, w[...], preferred_element_type=jnp.float32) + b[...]
  x = x * snorm[...]
  mu = jnp.mean(x, axis=0, keepdims=True)
  d = x - mu
  var = jnp.mean(d * d, axis=0, keepdims=True)
  x = gamma[...] * d * lax.rsqrt(var + 1e-5) + beta[...]
  x = jnp.maximum(x, 0.0)
  h = hin[...] + x
  hout_ref[...] = h
  xcat_ref[...] = h * outrs[...]


@functools.lru_cache(maxsize=None)
def _layer_call():
  return pl.pallas_call(
      _layer_body,
      out_shape=(
          jax.ShapeDtypeStruct((_N, _D), jnp.float32),
          jax.ShapeDtypeStruct((_N, _D), jnp.float32),
      ),
      compiler_params=pltpu.CompilerParams(
          vmem_limit_bytes=100 * 1024 * 1024),
  )


def _readout_body(h, gid, w1, b1, w2, b2, out_ref):
  ids = gid[...]
  gi = lax.broadcasted_iota(jnp.int32, (_N, _G), 1)
  m = (gi == ids).astype(jnp.float32)
  counts = jnp.sum(m, axis=0)
  sums = lax.dot_general(m, h[...], (((0,), (0,)), ((), ())),
                         preferred_element_type=jnp.float32)
  hg = sums / jnp.maximum(counts, 1.0)[:, None]
  y = jnp.maximum(
      jnp.dot(hg, w1[...], preferred_element_type=jnp.float32) + b1[...], 0.0)
  out_ref[...] = jnp.dot(y, w2[...],
                         preferred_element_type=jnp.float32) + b2[...]


@functools.lru_cache(maxsize=None)
def _readout_call():
  return pl.pallas_call(
      _readout_body,
      out_shape=jax.ShapeDtypeStruct((_G, _D), jnp.float32),
      compiler_params=pltpu.CompilerParams(
          vmem_limit_bytes=100 * 1024 * 1024),
  )


def kernel(nodes_feat, edge_index, edges_feat, nodes_num_norm_sqrt,
           edges_num_norm_sqrt, node_graph_ids, W_emb, b_emb, gcn_W, gcn_b,
           bn_gamma, bn_beta, W1, b1, W2, b2):
  del edges_feat, edges_num_norm_sqrt
  src = edge_index[0]
  dst = edge_index[1]

  npad_e = _EPAD - _E
  src_pad = jnp.concatenate([src, jnp.zeros((npad_e,), jnp.int32)])
  dst_pad = jnp.concatenate(
      [dst, jnp.full((npad_e,), _NPAD - 1, jnp.int32)])
  srcr = src_pad.reshape(_MCH, _MC)
  dstr = dst_pad.reshape(_MCH, _MC)

  srcd = jnp.concatenate(
      [src, jnp.full((npad_e,), _NPAD - 1, jnp.int32)]).reshape(_CHT, _CHUNK)
  dstd = dst_pad.reshape(_CHT, _CHUNK)
  din_p, dout_p = _deg_call()(srcd, dstd)
  h, xcat, inrs, outrs = _prep_call()(
      nodes_feat, W_emb, b_emb.reshape(1, _D), din_p, dout_p)

  for i in range(_NLAYERS):
    p = _msgpass_call()(xcat, srcr, dstr)
    h, xcat = _layer_call()(
        p, inrs, h, gcn_W[i], gcn_b[i].reshape(1, _D),
        nodes_num_norm_sqrt, bn_gamma[i].reshape(1, _D),
        bn_beta[i].reshape(1, _D), outrs)

  w2p = jnp.pad(W2, ((0, 0), (0, _D - W2.shape[1])))
  b2p = jnp.pad(b2, (0, _D - b2.shape[0])).reshape(1, _D)
  logits = _readout_call()(
      h, node_graph_ids.reshape(_N, 1), W1, b1.reshape(1, -1), w2p, b2p)
  return logits[:, :b2.shape[0]]


# speedup vs baseline: 1.4556x; 1.4556x over previous
import functools

import jax
import jax.numpy as jnp
from jax import lax
from jax.experimental import pallas as pl
from jax.experimental.pallas import tpu as pltpu
import jax.experimental.pallas.tpu_sc as plsc

_N = 10000
_E = 320000
_D = 128
_G = 128
_NLAYERS = 3

_NC = 2
_NS = 16
_CHUNK = 128
_NCHUNK = _E // 128
_NPAD = 10240
_RPT = _NPAD // _NS
_DSUB = 64
_EPAD = 327680
_CHT = _EPAD // _CHUNK
_CPT = _CHT // _NS


def _zero_vmem_2d(ref, rows, cols):
  def body(r, carry):
    for k in range(cols // 16):
      ref[r, pl.ds(k * 16, 16)] = jnp.zeros((16,), jnp.float32)
    return carry
  lax.fori_loop(0, rows, body, 0)


def _n_iters(w, nworkers):
  full = _NCHUNK // nworkers
  rem = _NCHUNK - full * nworkers
  return full + jnp.where(w < rem, 1, 0)


def _msgpass_body(x_hbm, srcr_hbm, dstr_hbm, out_hbm, *scr):
  rows = list(scr[0:8])
  didx = list(scr[8:16])
  sidx = scr[16]
  gsem = list(scr[17:25])
  dsem = list(scr[25:33])
  tsem = list(scr[33:37])

  c = lax.axis_index("c")
  s = lax.axis_index("s")

  _zero_vmem_2d(rows[0], _CHUNK, _DSUB)
  for k in range(_RPT // _CHUNK):
    pltpu.sync_copy(rows[0],
                    acc_ref(scr).at[pl.ds(s * _RPT + k * _CHUNK, _CHUNK)])

  pltpu.sync_copy(srcr_hbm.at[pl.ds((c * _CHT + s * _CPT), _CPT)], sidx)

  acc = acc_ref(scr)

  def gstart(j, q):
    pltpu.make_async_copy(x_hbm.at[sidx.at[j]], rows[q], gsem[q]).start()

  def gwait(j, q):
    pltpu.make_async_copy(x_hbm.at[sidx.at[j]], rows[q], gsem[q]).wait()

  def dstart(j, q):
    pltpu.make_async_copy(
        dstr_hbm.at[pl.ds(s * _CPT + j, 1)], didx[q], dsem[q]).start()

  def dwait(j, q):
    pltpu.make_async_copy(
        dstr_hbm.at[pl.ds(s * _CPT + j, 1)], didx[q], dsem[q]).wait()

  def sstart(q):
    pltpu.async_copy(rows[q], acc.at[didx[q].at[0]], tsem[q % 4], add=True)

  def swait(q):
    pltpu.make_async_copy(rows[q], acc.at[didx[q].at[0]], tsem[q % 4]).wait()

  for q in range(6):
    dstart(q, q)
    gstart(q, q)

  plsc.subcore_barrier()

  def iteration(j, q, first, last):
    gwait(j, q)
    dwait(j, q)
    if not first:
      swait((q + 6) % 8)
    sstart(q)
    if not last:
      gstart(j + 6, (q + 6) % 8)
      dstart(j + 6, (q + 6) % 8)

  for j in range(8):
    iteration(j, j % 8, j < 2, False)

  def body(p, carry):
    for q in range(8):
      iteration(8 * p + q, q, False, False)
    return carry

  lax.fori_loop(1, _CPT // 8 - 1, body, 0)

  for j in range(_CPT - 8, _CPT):
    iteration(j, j % 8, False, j >= _CPT - 6)
  for j in range(_CPT - 2, _CPT):
    swait(j % 8)

  plsc.subcore_barrier()
  for k in range(_RPT // _CHUNK):
    pltpu.sync_copy(acc.at[pl.ds(s * _RPT + k * _CHUNK, _CHUNK)], rows[k])
    pltpu.sync_copy(
        rows[k],
        out_hbm.at[pl.ds((c * _NS + s) * _RPT + k * _CHUNK, _CHUNK)])


def acc_ref(scr):
  return scr[37]


@functools.lru_cache(maxsize=None)
def _msgpass_call():
  mesh = plsc.VectorSubcoreMesh(
      core_axis_name="c", subcore_axis_name="s", num_cores=_NC)
  scratch = (
      [pltpu.VMEM((_CHUNK, _DSUB), jnp.float32)] * 8
      + [pltpu.VMEM((1, _CHUNK), jnp.int32)] * 8
      + [pltpu.VMEM((_CPT, _CHUNK), jnp.int32)]
      + [pltpu.SemaphoreType.DMA] * 20
      + [pltpu.VMEM_SHARED((_NPAD, _DSUB), jnp.float32)]
  )
  return pl.kernel(
      _msgpass_body,
      out_type=jax.ShapeDtypeStruct((_NC * _NPAD, _DSUB), jnp.float32),
      mesh=mesh,
      scratch_types=scratch,
      compiler_params=pltpu.CompilerParams(use_tc_tiling_on_sc=False),
  )


def _deg_body(srcd_hbm, dstr_hbm, oin_hbm, oout_hbm,
              sidx, didx, hin, hout):
  c = lax.axis_index("c")
  s = lax.axis_index("s")
  w = c * _NS + s

  def zbody(r, carry):
    hin[pl.ds(r * 16, 16)] = jnp.zeros((16,), jnp.float32)
    hout[pl.ds(r * 16, 16)] = jnp.zeros((16,), jnp.float32)
    return carry
  lax.fori_loop(0, _NPAD // 16, zbody, 0)

  cpw = _CHT // 32
  pltpu.sync_copy(srcd_hbm.at[pl.ds(w * cpw, cpw)], sidx)
  pltpu.sync_copy(dstr_hbm.at[pl.ds(w * cpw, cpw)], didx)

  ones16 = jnp.ones((16,), jnp.float32)

  def step(j, carry):
    for k in range(_CHUNK // 16):
      plsc.addupdate_scatter(hout, [sidx[j, pl.ds(k * 16, 16)]], ones16)
      plsc.addupdate_scatter(hin, [didx[j, pl.ds(k * 16, 16)]], ones16)
    return carry

  lax.fori_loop(0, cpw, step, 0)

  pltpu.sync_copy(hin, oin_hbm.at[w])
  pltpu.sync_copy(hout, oout_hbm.at[w])


@functools.lru_cache(maxsize=None)
def _deg_call():
  mesh = plsc.VectorSubcoreMesh(
      core_axis_name="c", subcore_axis_name="s", num_cores=_NC)
  return pl.kernel(
      _deg_body,
      out_type=(
          jax.ShapeDtypeStruct((2 * _NS, _NPAD), jnp.float32),
          jax.ShapeDtypeStruct((2 * _NS, _NPAD), jnp.float32),
      ),
      mesh=mesh,
      scratch_types=[
          pltpu.VMEM((_CHT // 32, _CHUNK), jnp.int32),
          pltpu.VMEM((_CHT // 32, _CHUNK), jnp.int32),
          pltpu.VMEM((_NPAD,), jnp.float32),
          pltpu.VMEM((_NPAD,), jnp.float32),
      ],
      compiler_params=pltpu.CompilerParams(needs_layout_passes=False),
  )


def _prep_body(nodes, wemb, bemb, din_p, dout_p,
               h0_ref, xcat_ref, inrs_ref, outrs_ref):
  h0 = jnp.dot(nodes[...], wemb[...],
               preferred_element_type=jnp.float32) + bemb[...]
  din = jnp.sum(din_p[...], axis=0)[0:_N, None]
  dout = jnp.sum(dout_p[...], axis=0)[0:_N, None]
  inrs = lax.rsqrt(jnp.maximum(din, 1.0))
  outrs = lax.rsqrt(jnp.maximum(dout, 1.0))
  x0 = h0 * outrs
  h0_ref[...] = h0
  xcat_ref[0:_N] = x0[:, 0:_DSUB]
  xcat_ref[_N:2 * _N] = x0[:, _DSUB:_D]
  inrs_ref[...] = inrs
  outrs_ref[...] = outrs


@functools.lru_cache(maxsize=None)
def _prep_call():
  return pl.pallas_call(
      _prep_body,
      out_shape=(
          jax.ShapeDtypeStruct((_N, _D), jnp.float32),
          jax.ShapeDtypeStruct((2 * _N, _DSUB), jnp.float32),
          jax.ShapeDtypeStruct((_N, 1), jnp.float32),
          jax.ShapeDtypeStruct((_N, 1), jnp.float32),
      ),
      compiler_params=pltpu.CompilerParams(
          vmem_limit_bytes=100 * 1024 * 1024),
  )


def _layer_body(p, inrs, hin, w, b, snorm, gamma, beta, outrs,
                hout_ref, xcat_ref):
  agg = jnp.concatenate(
      [p[0:_N], p[_NPAD:_NPAD + _N]], axis=1)
  agg = agg * inrs[...]
  x = jnp.dot(agg, w[...], preferred_element_type=jnp.float32) + b[...]
  x = x * snorm[...]
  mu = jnp.mean(x, axis=0, keepdims=True)
  d = x - mu
  var = jnp.mean(d * d, axis=0, keepdims=True)
  x = gamma[...] * d * lax.rsqrt(var + 1e-5) + beta[...]
  x = jnp.maximum(x, 0.0)
  h = hin[...] + x
  hout_ref[...] = h
  xn = h * outrs[...]
  xcat_ref[0:_N] = xn[:, 0:_DSUB]
  xcat_ref[_N:2 * _N] = xn[:, _DSUB:_D]


@functools.lru_cache(maxsize=None)
def _layer_call():
  return pl.pallas_call(
      _layer_body,
      out_shape=(
          jax.ShapeDtypeStruct((_N, _D), jnp.float32),
          jax.ShapeDtypeStruct((2 * _N, _DSUB), jnp.float32),
      ),
      compiler_params=pltpu.CompilerParams(
          vmem_limit_bytes=100 * 1024 * 1024),
  )


def _readout_body(h, gid, w1, b1, w2, b2, out_ref):
  ids = gid[...]
  gi = lax.broadcasted_iota(jnp.int32, (_N, _G), 1)
  m = (gi == ids).astype(jnp.float32)
  counts = jnp.sum(m, axis=0)
  sums = lax.dot_general(m, h[...], (((0,), (0,)), ((), ())),
                         preferred_element_type=jnp.float32)
  hg = sums / jnp.maximum(counts, 1.0)[:, None]
  y = jnp.maximum(
      jnp.dot(hg, w1[...], preferred_element_type=jnp.float32) + b1[...], 0.0)
  out_ref[...] = jnp.dot(y, w2[...],
                         preferred_element_type=jnp.float32) + b2[...]


@functools.lru_cache(maxsize=None)
def _readout_call():
  return pl.pallas_call(
      _readout_body,
      out_shape=jax.ShapeDtypeStruct((_G, _D), jnp.float32),
      compiler_params=pltpu.CompilerParams(
          vmem_limit_bytes=100 * 1024 * 1024),
  )


def kernel(nodes_feat, edge_index, edges_feat, nodes_num_norm_sqrt,
           edges_num_norm_sqrt, node_graph_ids, W_emb, b_emb, gcn_W, gcn_b,
           bn_gamma, bn_beta, W1, b1, W2, b2):
  del edges_feat, edges_num_norm_sqrt
  src = edge_index[0]
  dst = edge_index[1]

  npad_e = _EPAD - _E
  src_pad = jnp.concatenate([src, jnp.zeros((npad_e,), jnp.int32)])
  dst_pad = jnp.concatenate(
      [dst, jnp.full((npad_e,), _NPAD - 1, jnp.int32)])
  srcr = jnp.concatenate([src_pad, src_pad + _N]).reshape(2 * _CHT, _CHUNK)
  dstr = dst_pad.reshape(_CHT, _CHUNK)

  srcd = jnp.concatenate(
      [src, jnp.full((npad_e,), _NPAD - 1, jnp.int32)]).reshape(_CHT, _CHUNK)
  din_p, dout_p = _deg_call()(srcd, dstr)
  h, xcat, inrs, outrs = _prep_call()(
      nodes_feat, W_emb, b_emb.reshape(1, _D), din_p, dout_p)

  for i in range(_NLAYERS):
    p = _msgpass_call()(xcat, srcr, dstr)
    h, xcat = _layer_call()(
        p, inrs, h, gcn_W[i], gcn_b[i].reshape(1, _D),
        nodes_num_norm_sqrt, bn_gamma[i].reshape(1, _D),
        bn_beta[i].reshape(1, _D), outrs)

  w2p = jnp.pad(W2, ((0, 0), (0, _D - W2.shape[1])))
  b2p = jnp.pad(b2, (0, _D - b2.shape[0])).reshape(1, _D)
  logits = _readout_call()(
      h, node_graph_ids.reshape(_N, 1), W1, b1.reshape(1, -1), w2p, b2p)
  return logits[:, :b2.shape[0]]
